# baseline (speedup 1.0000x reference)
import functools

import jax
import jax.numpy as jnp
from jax.experimental import pallas as pl

_B, _C, _H, _W = 8, 96, 128, 128
_E = 8


def _moe_kernel(x_ref, k_ref, gw_ref, gb_ref, ew_ref, eb_ref, o_ref):
    b = pl.program_id(0)
    x = x_ref[...]
    pooled = jnp.sum(x, axis=(2, 3)) * (1.0 / (_H * _W))
    logits = jnp.sum(pooled.T * gw_ref[...], axis=0)[None, :] + gb_ref[...]
    w = jax.nn.softmax(logits, axis=-1)

    idx = jax.lax.broadcasted_iota(jnp.int32, (1, _E), 1)
    i1 = jnp.argmax(w, axis=1)[:, None]
    mask1 = idx == i1
    w_rest = jnp.where(mask1, -jnp.inf, w)
    i2 = jnp.argmax(w_rest, axis=1)[:, None]
    g = jnp.where(mask1 | (idx == i2), w, 0.0)

    gt = g.T
    scale = 1.0 + jnp.sum(gt * ew_ref[...], axis=0)[None, :]
    k_row = k_ref[pl.ds(b, 1), :]
    bias = jnp.sum(gt * eb_ref[...], axis=0)[None, :] + jnp.sum(g) * k_row
    o_ref[...] = x * scale[:, :, None, None] + bias[:, :, None, None]


@jax.jit
def kernel(inputs, k, gate_w, gate_b, expert_w, expert_b):
    k2 = k.reshape(_B, _C)
    gb2 = gate_b.reshape(1, _E)
    return pl.pallas_call(
        _moe_kernel,
        grid=(_B,),
        in_specs=[
            pl.BlockSpec((1, _C, _H, _W), lambda b: (b, 0, 0, 0)),
            pl.BlockSpec((_B, _C), lambda b: (0, 0)),
            pl.BlockSpec((_C, _E), lambda b: (0, 0)),
            pl.BlockSpec((1, _E), lambda b: (0, 0)),
            pl.BlockSpec((_E, _C), lambda b: (0, 0)),
            pl.BlockSpec((_E, _C), lambda b: (0, 0)),
        ],
        out_specs=pl.BlockSpec((1, _C, _H, _W), lambda b: (b, 0, 0, 0)),
        out_shape=jax.ShapeDtypeStruct((_B, _C, _H, _W), inputs.dtype),
    )(inputs, k2, gate_w, gb2, expert_w, expert_b)


# speedup vs baseline: 1.0018x; 1.0018x over previous
import functools

import jax
import jax.numpy as jnp
from jax.experimental import pallas as pl
from jax.experimental.pallas import tpu as pltpu

_B, _C, _H, _W = 8, 96, 128, 128
_E = 8


def _moe_kernel(x_ref, k_ref, gw_ref, gb_ref, ew_ref, eb_ref, o_ref,
                img_ref, co_ref):
    s = pl.program_id(0)
    slot = jax.lax.rem(s, 2)
    prev = jax.lax.rem(s + 1, 2)

    @pl.when(s < _B)
    def _reduce_and_gate():
        x = x_ref[...]
        img_ref[pl.ds(slot, 1)] = x
        pooled = jnp.sum(x, axis=(2, 3)) * (1.0 / (_H * _W))
        logits = (
            jnp.dot(pooled, gw_ref[...], preferred_element_type=jnp.float32)
            + gb_ref[...]
        )
        w = jax.nn.softmax(logits, axis=-1)

        idx = jax.lax.broadcasted_iota(jnp.int32, (1, _E), 1)
        i1 = jnp.argmax(w, axis=1)[:, None]
        mask1 = idx == i1
        w_rest = jnp.where(mask1, -jnp.inf, w)
        i2 = jnp.argmax(w_rest, axis=1)[:, None]
        g = jnp.where(mask1 | (idx == i2), w, 0.0)

        scale = 1.0 + jnp.dot(g, ew_ref[...],
                              preferred_element_type=jnp.float32)
        k_row = k_ref[pl.ds(s, 1), :]
        bias = (
            jnp.dot(g, eb_ref[...], preferred_element_type=jnp.float32)
            + jnp.sum(g) * k_row
        )
        co_ref[pl.ds(slot, 1), 0, :] = scale
        co_ref[pl.ds(slot, 1), 1, :] = bias

    @pl.when(s > 0)
    def _affine():
        xp = img_ref[pl.ds(prev, 1)]
        scale = co_ref[pl.ds(prev, 1), 0, :]
        bias = co_ref[pl.ds(prev, 1), 1, :]
        o_ref[...] = xp * scale[:, :, None, None] + bias[:, :, None, None]


@jax.jit
def kernel(inputs, k, gate_w, gate_b, expert_w, expert_b):
    k2 = k.reshape(_B, _C)
    gb2 = gate_b.reshape(1, _E)
    return pl.pallas_call(
        _moe_kernel,
        grid=(_B + 1,),
        in_specs=[
            pl.BlockSpec((1, _C, _H, _W),
                         lambda s: (jnp.minimum(s, _B - 1), 0, 0, 0)),
            pl.BlockSpec((_B, _C), lambda s: (0, 0)),
            pl.BlockSpec((_C, _E), lambda s: (0, 0)),
            pl.BlockSpec((1, _E), lambda s: (0, 0)),
            pl.BlockSpec((_E, _C), lambda s: (0, 0)),
            pl.BlockSpec((_E, _C), lambda s: (0, 0)),
        ],
        out_specs=pl.BlockSpec((1, _C, _H, _W),
                               lambda s: (jnp.maximum(s - 1, 0), 0, 0, 0)),
        out_shape=jax.ShapeDtypeStruct((_B, _C, _H, _W), inputs.dtype),
        scratch_shapes=[
            pltpu.VMEM((2, _C, _H, _W), jnp.float32),
            pltpu.VMEM((2, 2, _C), jnp.float32),
        ],
    )(inputs, k2, gate_w, gb2, expert_w, expert_b)


# speedup vs baseline: 1.0039x; 1.0021x over previous
import functools

import jax
import jax.numpy as jnp
from jax.experimental import pallas as pl

_B, _C, _H, _W = 8, 96, 128, 128
_E = 8


def _moe_kernel(x_ref, k_ref, gw_ref, gb_ref, ew_ref, eb_ref, o_ref):
    b = pl.program_id(0)
    x = x_ref[...]
    pooled = jnp.sum(x, axis=(2, 3)) * (1.0 / (_H * _W))
    logits = (
        jnp.dot(pooled, gw_ref[...], preferred_element_type=jnp.float32)
        + gb_ref[...]
    )
    e = jnp.exp(logits - jnp.max(logits))
    denom = jnp.sum(e)

    ecol = e.reshape(_E, 1)
    erow = jnp.broadcast_to(e, (_E, _E))
    jidx = jax.lax.broadcasted_iota(jnp.int32, (_E, _E), 1)
    iidx = jax.lax.broadcasted_iota(jnp.int32, (_E, _E), 0)
    beats = (erow > ecol) | ((erow == ecol) & (jidx < iidx))
    rank = jnp.sum(beats.astype(jnp.float32), axis=1)[None, :]
    g = jnp.where(rank < 2.0, e, 0.0)

    inv = 1.0 / denom
    scale = 1.0 + inv * jnp.dot(g, ew_ref[...],
                                preferred_element_type=jnp.float32)
    k_row = k_ref[pl.ds(b, 1), :]
    bias = inv * (
        jnp.dot(g, eb_ref[...], preferred_element_type=jnp.float32)
        + jnp.sum(g) * k_row
    )
    o_ref[...] = x * scale[:, :, None, None] + bias[:, :, None, None]


@jax.jit
def kernel(inputs, k, gate_w, gate_b, expert_w, expert_b):
    k2 = k.reshape(_B, _C)
    gb2 = gate_b.reshape(1, _E)
    return pl.pallas_call(
        _moe_kernel,
        grid=(_B,),
        in_specs=[
            pl.BlockSpec((1, _C, _H, _W), lambda b: (b, 0, 0, 0)),
            pl.BlockSpec((_B, _C), lambda b: (0, 0)),
            pl.BlockSpec((_C, _E), lambda b: (0, 0)),
            pl.BlockSpec((1, _E), lambda b: (0, 0)),
            pl.BlockSpec((_E, _C), lambda b: (0, 0)),
            pl.BlockSpec((_E, _C), lambda b: (0, 0)),
        ],
        out_specs=pl.BlockSpec((1, _C, _H, _W), lambda b: (b, 0, 0, 0)),
        out_shape=jax.ShapeDtypeStruct((_B, _C, _H, _W), inputs.dtype),
    )(inputs, k2, gate_w, gb2, expert_w, expert_b)
